# baseline (speedup 1.0000x reference)
import numpy as np
import jax
import jax.numpy as jnp
from jax.experimental import pallas as pl
from jax.experimental.pallas import tpu as pltpu

_B = 8
_CH = 8
_TAPS9 = [(dh, dw) for dh in (-1, 0, 1) for dw in (-1, 0, 1)]
_BF = jnp.bfloat16


def _tap_masks_np(S, B):
    P = S * S
    m = np.zeros((9, 1, P), np.float32)
    for t, (dh, dw) in enumerate(_TAPS9):
        for h in range(S):
            for w in range(S):
                if 0 <= h + dh < S and 0 <= w + dw < S:
                    m[t, 0, h * S + w] = 1.0
    return np.tile(m, (1, 1, B))


def _pool_select_np(S, B):
    So = S // 2
    g = np.zeros((S * S, So * So), np.float32)
    for ho in range(So):
        for wo in range(So):
            g[(2 * ho) * S + 2 * wo, ho * So + wo] = 1.0
    return np.kron(np.eye(B, dtype=np.float32), g)


def _upsample_scatter_np(S, B):
    p = np.zeros((4, S * S, 4 * S * S), np.float32)
    for kh in range(2):
        for kw in range(2):
            t = kh * 2 + kw
            for h in range(S):
                for w in range(S):
                    p[t, h * S + w, (2 * h + kh) * (2 * S) + (2 * w + kw)] = 1.0
    return np.concatenate(
        [np.kron(np.eye(B, dtype=np.float32), p[t]) for t in range(4)], axis=0)


def _conv3x3_relu(xb, S, w_all, layer, cin, cout, mslab, ball, bj, m_ref):
    y = jnp.dot(w_all[layer][:, :cin], xb,
                preferred_element_type=jnp.float32)
    acc = None
    for t, (dh, dw) in enumerate(_TAPS9):
        off = dh * S + dw
        s = y[t * mslab:t * mslab + cout]
        if off:
            s = jnp.roll(s, -off, axis=1)
        if dh != 0 or dw != 0:
            s = s * m_ref[t]
        acc = s if acc is None else acc + s
    return jnp.maximum(acc + ball[:cout, bj:bj + 1], 0.0)


def _maxpool2x2(e, S, g_ref, nchunk=1):
    t1 = jnp.roll(e, -1, axis=1)
    t2 = jnp.roll(e, -S, axis=1)
    t3 = jnp.roll(e, -(S + 1), axis=1)
    m = jnp.maximum(jnp.maximum(e, t1), jnp.maximum(t2, t3)).astype(_BF)
    if nchunk == 1:
        return jnp.dot(m, g_ref[...], preferred_element_type=jnp.float32)
    w = m.shape[1] // nchunk
    return jnp.concatenate(
        [jnp.dot(m[:, c * w:(c + 1) * w], g_ref[...],
                 preferred_element_type=jnp.float32) for c in range(nchunk)],
        axis=1)


def _conv_transpose2x2(xcat, ct, layer, cin, cout, ball, bj, p_ref, nchunk=1):
    b_ref = ball[:cout, bj:bj + 1]
    parts = [jnp.dot(ct[layer, t][:cout, :cin], xcat,
                     preferred_element_type=jnp.float32)
             for t in range(4)]
    Lin = xcat.shape[1]
    w = Lin // nchunk
    outs = []
    for c in range(nchunk):
        al = jnp.concatenate([pt[:, c * w:(c + 1) * w] for pt in parts],
                             axis=1).astype(_BF)
        outs.append(jnp.dot(al, p_ref[...], preferred_element_type=jnp.float32))
    o = outs[0] if nchunk == 1 else jnp.concatenate(outs, axis=1)
    return o + b_ref


def _unet_kernel(x_ref, m16, m8, m4, m2, wA, wB, ball,
                 g1, g2, g3, ct, p2, p4, p8, ow, o_ref):
    C, P = x_ref.shape[1], x_ref.shape[2]

    def both(f):
        return [f(i) for i in range(_CH)]

    x = both(lambda i: jnp.concatenate(
        [x_ref[i * _B + b].astype(_BF) for b in range(_B)], axis=1))

    t = both(lambda i: _conv3x3_relu(x[i], 16, wA, 0, 64, 32, 32, ball, 0, m16).astype(_BF))
    e1 = both(lambda i: _conv3x3_relu(t[i], 16, wA, 1, 32, 32, 32, ball, 1, m16))
    e1b = [v.astype(_BF) for v in e1]
    p1 = both(lambda i: _maxpool2x2(e1[i], 16, g1).astype(_BF))
    t = both(lambda i: _conv3x3_relu(p1[i], 8, wB, 0, 32, 16, 16, ball, 2, m8).astype(_BF))
    e2 = both(lambda i: _conv3x3_relu(t[i], 8, wB, 1, 16, 16, 16, ball, 3, m8))
    e2b = [v.astype(_BF) for v in e2]
    p2v = both(lambda i: _maxpool2x2(e2[i], 8, g2).astype(_BF))
    t = both(lambda i: _conv3x3_relu(p2v[i], 4, wB, 2, 16, 8, 16, ball, 4, m4).astype(_BF))
    e3 = both(lambda i: _conv3x3_relu(t[i], 4, wB, 3, 8, 8, 16, ball, 5, m4))
    e3b = [v.astype(_BF) for v in e3]
    p3v = both(lambda i: _maxpool2x2(e3[i], 4, g3).astype(_BF))
    t = both(lambda i: _conv3x3_relu(p3v[i], 2, wB, 4, 8, 8, 16, ball, 6, m2).astype(_BF))
    bn = both(lambda i: _conv3x3_relu(t[i], 2, wB, 5, 8, 8, 16, ball, 7, m2))

    u3 = both(lambda i: _conv_transpose2x2(
        bn[i].astype(_BF), ct, 0, 8, 8, ball, 8, p2))
    u2 = both(lambda i: _conv_transpose2x2(
        jnp.concatenate([u3[i].astype(_BF), e3b[i]], axis=0),
        ct, 1, 16, 16, ball, 9, p4))
    u1 = both(lambda i: _conv_transpose2x2(
        jnp.concatenate([u2[i].astype(_BF), e2b[i]], axis=0),
        ct, 2, 32, 32, ball, 10, p8, nchunk=4))
    out = both(lambda i: jnp.dot(
        ow[...], jnp.concatenate([u1[i].astype(_BF), e1b[i]], axis=0),
        preferred_element_type=jnp.float32) + ball[:, 11:12])
    for i in range(_CH):
        for b in range(_B):
            o_ref[i * _B + b] = out[i][:, b * P:(b + 1) * P]


def _wstack(ws, kpad, mpad):
    padded = [jnp.pad(w, ((0, 0), (0, 0), (0, kpad - w.shape[2]),
                          (0, mpad - w.shape[3]))) for w in ws]
    s = jnp.stack(padded)
    return jnp.transpose(s, (0, 1, 2, 4, 3)).reshape(
        len(ws), 9 * mpad, kpad).astype(_BF)


def _ctstack(ws):
    padded = [jnp.pad(w, ((0, 0), (0, 0), (0, 32 - w.shape[2]),
                          (0, 32 - w.shape[3]))) for w in ws]
    s = jnp.stack(padded)
    return jnp.transpose(s, (0, 1, 2, 4, 3)).reshape(len(ws), 4, 32, 32).astype(_BF)


def _bstack(bs):
    return jnp.stack([jnp.pad(b, (0, 64 - b.shape[0])) for b in bs]).T


def kernel(enc1_w1, enc1_b1, enc1_w2, enc1_b2, enc2_w1, enc2_b1, enc2_w2,
           enc2_b2, enc3_w1, enc3_b1, enc3_w2, enc3_b2, bn_w1, bn_b1, bn_w2,
           bn_b2, up3_w, up3_b, up2_w, up2_b, up1_w, up1_b, out_w, out_b, x):
    N, C, H, W = x.shape
    P = H * W
    B = _B
    f32a = lambda a: jnp.asarray(a, dtype=jnp.float32)
    bf = lambda a: jnp.asarray(a, dtype=_BF)

    consts = (
        f32a(_tap_masks_np(16, B)), f32a(_tap_masks_np(8, B)),
        f32a(_tap_masks_np(4, B)), f32a(_tap_masks_np(2, B)),
        _wstack([enc1_w1, enc1_w2], 64, 32),
        _wstack([enc2_w1, enc2_w2, enc3_w1, enc3_w2, bn_w1, bn_w2], 32, 16),
        _bstack([enc1_b1, enc1_b2, enc2_b1, enc2_b2, enc3_b1, enc3_b2,
                 bn_b1, bn_b2, up3_b, up2_b, up1_b, out_b]),
        bf(_pool_select_np(16, B)), bf(_pool_select_np(8, B)),
        bf(_pool_select_np(4, B)),
        _ctstack([up3_w, up2_w, up1_w]),
        bf(_upsample_scatter_np(2, B)), bf(_upsample_scatter_np(4, B)),
        bf(_upsample_scatter_np(8, 2)),
        out_w.T.astype(_BF),
    )

    x2 = x.reshape(N, C, P)
    G = _CH * B

    def _call(xs, *cs):
        Ns = xs.shape[0]
        in_specs = [pl.BlockSpec((G, C, P), lambda n: (n, 0, 0))]
        for a in cs:
            in_specs.append(pl.BlockSpec(a.shape, lambda n, _nd=a.ndim: (0,) * _nd))
        return pl.pallas_call(
            _unet_kernel,
            out_shape=jax.ShapeDtypeStruct((Ns, C, P), jnp.float32),
            grid=(Ns // G,),
            in_specs=in_specs,
            out_specs=pl.BlockSpec((G, C, P), lambda n: (n, 0, 0)),
            compiler_params=pltpu.CompilerParams(
                dimension_semantics=("parallel",),
                vmem_limit_bytes=64 * 1024 * 1024),
        )(xs, *cs)

    out = _call(x2, *consts)
    return out.reshape(N, C, H, W)
